# baseline (speedup 1.0000x reference)
import jax
import jax.numpy as jnp
from jax.experimental import pallas as pl
from jax.experimental.pallas import tpu as pltpu

SEQ_BLK = 512
D_BLK = 512


def _pe_add_kernel(x_ref, pe_ref, o_ref):
    o_ref[...] = x_ref[...] + pe_ref[...][:, None, :]


def kernel(x, pe_weight):
    seq_len, batch, d_model = x.shape
    grid = (seq_len // SEQ_BLK, d_model // D_BLK)
    return pl.pallas_call(
        _pe_add_kernel,
        grid=grid,
        in_specs=[
            pl.BlockSpec((SEQ_BLK, batch, D_BLK), lambda i, j: (i, 0, j)),
            pl.BlockSpec((SEQ_BLK, D_BLK), lambda i, j: (i, j)),
        ],
        out_specs=pl.BlockSpec((SEQ_BLK, batch, D_BLK), lambda i, j: (i, 0, j)),
        out_shape=jax.ShapeDtypeStruct((seq_len, batch, d_model), x.dtype),
        compiler_params=pltpu.CompilerParams(
            dimension_semantics=("parallel", "parallel"),
        ),
    )(x, pe_weight)
